# baseline (speedup 1.0000x reference)
import functools

import jax
import jax.numpy as jnp
from jax import lax
from jax.experimental import pallas as pl
from jax.experimental.pallas import tpu as pltpu
from jax.experimental.pallas import tpu_sc as plsc

EMB = 32
BATCH = 16384
FIELDS = 26

NUM_CORES = 2
NUM_SUBCORES = 16
NW = NUM_CORES * NUM_SUBCORES
BPW = BATCH // NW
SB = 128
NSB = BPW // SB

_mesh = plsc.VectorSubcoreMesh(core_axis_name="c", subcore_axis_name="s")


@functools.partial(
    pl.kernel,
    mesh=_mesh,
    compiler_params=pltpu.CompilerParams(
        use_tc_tiling_on_sc=True, needs_layout_passes=False
    ),
    out_type=jax.ShapeDtypeStruct((FIELDS, EMB, BATCH), jnp.float32),
    scratch_types=[
        pltpu.VMEM((FIELDS, BPW), jnp.int32),
        pltpu.VMEM((SB,), jnp.int32),
        pltpu.VMEM((SB, 128), jnp.float32),
        pltpu.VMEM((EMB, BPW), jnp.float32),
        pltpu.SemaphoreType.DMA,
    ],
)
def _emb_lookup(xt_hbm, tab_hbm, out_hbm, idx_v, bkt_v, rows_v, obuf, sem):
    wid = lax.axis_index("s") * NUM_CORES + lax.axis_index("c")
    b0 = wid * BPW

    pltpu.sync_copy(xt_hbm.at[:, pl.ds(b0, BPW)], idx_v)

    def f_body(f, carry):
        def sb_body(sb, carry2):
            c0 = sb * SB
            for g in range(SB // 16):
                v = idx_v[f, pl.ds(c0 + g * 16, 16)]
                bkt_v[pl.ds(g * 16, 16)] = lax.shift_right_logical(v, 2)
            pltpu.async_copy(tab_hbm.at[bkt_v], rows_v, sem).wait()
            for bg in range(SB // 16):
                r_idx = lax.iota(jnp.int32, 16) + bg * 16
                q = idx_v[f, pl.ds(c0 + bg * 16, 16)] & 3
                col = q * EMB
                for e in range(EMB):
                    val = plsc.load_gather(rows_v, [r_idx, col])
                    obuf[e, pl.ds(c0 + bg * 16, 16)] = val
                    col = col + 1
            return carry2

        lax.fori_loop(0, NSB, sb_body, 0)
        pltpu.sync_copy(obuf, out_hbm.at[f, :, pl.ds(b0, BPW)])
        return carry

    lax.fori_loop(0, FIELDS, f_body, 0)


def kernel(x, table):
    xt = x.T
    tabr = table.reshape(250000, 128)
    out = _emb_lookup(xt, tabr)
    return out.transpose(2, 0, 1)


# speedup vs baseline: 1.1694x; 1.1694x over previous
import functools

import jax
import jax.numpy as jnp
from jax import lax
from jax.experimental import pallas as pl
from jax.experimental.pallas import tpu as pltpu
from jax.experimental.pallas import tpu_sc as plsc

EMB = 32
BATCH = 16384
FIELDS = 26

NUM_CORES = 2
NUM_SUBCORES = 16
NW = NUM_CORES * NUM_SUBCORES
BPW = BATCH // NW
SB = 128
NSB = BPW // SB

_mesh = plsc.VectorSubcoreMesh(core_axis_name="c", subcore_axis_name="s")


@functools.partial(
    pl.kernel,
    mesh=_mesh,
    compiler_params=pltpu.CompilerParams(
        use_tc_tiling_on_sc=True, needs_layout_passes=False
    ),
    out_type=jax.ShapeDtypeStruct((FIELDS, EMB, BATCH), jnp.float32),
    scratch_types=[
        pltpu.VMEM((FIELDS, BPW), jnp.int32),
        pltpu.VMEM((SB,), jnp.int32),
        pltpu.VMEM((SB, 128), jnp.float32),
        pltpu.VMEM((EMB, BPW), jnp.float32),
        pltpu.SemaphoreType.DMA,
    ],
)
def _emb_lookup(xt_hbm, tab_hbm, out_hbm, idx_v, bkt_v, rows_v, obuf, sem):
    wid = lax.axis_index("s") * NUM_CORES + lax.axis_index("c")
    b0 = wid * BPW

    pltpu.sync_copy(xt_hbm.at[:, pl.ds(b0, BPW)], idx_v)

    def f_body(f, carry):
        for sb in range(NSB):
            c0 = sb * SB
            for g in range(SB // 16):
                v = idx_v[f, pl.ds(c0 + g * 16, 16)]
                bkt_v[pl.ds(g * 16, 16)] = lax.shift_right_logical(v, 2)
            pltpu.async_copy(tab_hbm.at[bkt_v], rows_v, sem).wait()
            for bg in range(SB // 16):
                r_idx = lax.iota(jnp.int32, 16) + bg * 16
                q = idx_v[f, pl.ds(c0 + bg * 16, 16)] & 3
                dst0 = c0 + bg * 16

                @plsc.parallel_loop(0, EMB, 1, unroll=8, carry=q * EMB)
                def _t(e, col):
                    val = plsc.load_gather(rows_v, [r_idx, col])
                    obuf[e, pl.ds(dst0, 16)] = val
                    return col + 1

        pltpu.sync_copy(obuf, out_hbm.at[f, :, pl.ds(b0, BPW)])
        return carry

    lax.fori_loop(0, FIELDS, f_body, 0)


def kernel(x, table):
    xt = x.T
    tabr = table.reshape(250000, 128)
    out = _emb_lookup(xt, tabr)
    return out.transpose(2, 0, 1)


# speedup vs baseline: 1.4330x; 1.2254x over previous
import functools

import jax
import jax.numpy as jnp
from jax import lax
from jax.experimental import pallas as pl
from jax.experimental.pallas import tpu as pltpu
from jax.experimental.pallas import tpu_sc as plsc

EMB = 32
BATCH = 16384
FIELDS = 26

NUM_CORES = 2
NUM_SUBCORES = 16
NW = NUM_CORES * NUM_SUBCORES
BPW = BATCH // NW
SB = 128
NSB = BPW // SB

_mesh = plsc.VectorSubcoreMesh(core_axis_name="c", subcore_axis_name="s")


@functools.partial(
    pl.kernel,
    mesh=_mesh,
    compiler_params=pltpu.CompilerParams(
        use_tc_tiling_on_sc=True, needs_layout_passes=False
    ),
    out_type=jax.ShapeDtypeStruct((FIELDS, EMB, BATCH), jnp.float32),
    scratch_types=[
        pltpu.VMEM((FIELDS, BPW), jnp.int32),
        pltpu.VMEM((2, SB), jnp.int32),
        pltpu.VMEM((2, SB, 128), jnp.float32),
        pltpu.VMEM((2, EMB, BPW), jnp.float32),
        pltpu.SemaphoreType.DMA,
        pltpu.SemaphoreType.DMA,
    ],
)
def _emb_lookup(xt_hbm, tab_hbm, out_hbm, idx_v, bkt_v, rows_v, obuf, gsem, osem):
    wid = lax.axis_index("s") * NUM_CORES + lax.axis_index("c")
    b0 = wid * BPW

    pltpu.sync_copy(xt_hbm.at[:, pl.ds(b0, BPW)], idx_v)

    def compute_bkt(f, sb, slot):
        c0 = sb * SB
        for g in range(SB // 16):
            v = idx_v[f, pl.ds(c0 + g * 16, 16)]
            bkt_v[slot, pl.ds(g * 16, 16)] = lax.shift_right_logical(v, 2)

    def gstart(slot):
        pltpu.make_async_copy(
            tab_hbm.at[bkt_v.at[slot]], rows_v.at[slot], gsem
        ).start()

    def gwait(slot):
        pltpu.make_async_copy(
            tab_hbm.at[bkt_v.at[0]], rows_v.at[slot], gsem
        ).wait()

    def ostart(f, oslot):
        pltpu.make_async_copy(
            obuf.at[oslot], out_hbm.at[f, :, pl.ds(b0, BPW)], osem
        ).start()

    def owait(oslot):
        pltpu.make_async_copy(
            obuf.at[oslot], out_hbm.at[0, :, pl.ds(b0, BPW)], osem
        ).wait()

    compute_bkt(0, 0, 0)
    gstart(0)

    def f_body(f, carry):
        oslot = lax.rem(f, 2)
        @pl.when(f >= 2)
        def _():
            owait(oslot)

        for sb in range(NSB):
            slot = sb % 2
            gwait(slot)
            if sb + 1 < NSB:
                compute_bkt(f, sb + 1, 1 - slot)
                gstart(1 - slot)
            else:
                @pl.when(f + 1 < FIELDS)
                def _():
                    compute_bkt(f + 1, 0, 1 - slot)
                    gstart(1 - slot)

            for bg in range(SB // 16):
                r_idx = lax.iota(jnp.int32, 16) + bg * 16
                q = idx_v[f, pl.ds(sb * SB + bg * 16, 16)] & 3
                dst0 = sb * SB + bg * 16

                @plsc.parallel_loop(0, EMB, 1, unroll=8, carry=q * EMB)
                def _t(e, col):
                    val = plsc.load_gather(rows_v.at[slot], [r_idx, col])
                    obuf[oslot, e, pl.ds(dst0, 16)] = val
                    return col + 1

        ostart(f, oslot)
        return carry

    lax.fori_loop(0, FIELDS, f_body, 0)
    owait(0)
    owait(1)


def kernel(x, table):
    xt = x.T
    tabr = table.reshape(250000, 128)
    out = _emb_lookup(xt, tabr)
    return out.transpose(2, 0, 1)
